# baseline (speedup 1.0000x reference)
import jax
import jax.numpy as jnp
from jax import lax
from jax.experimental import pallas as pl
from jax.experimental.pallas import tpu as pltpu

_VOCAB = 100000
_EMBED = 64
_B = 16
_L = 16
_TOKENS = _B * _L
_VT = 4096
_NT = 25
_TAIL = _VOCAB - (_NT - 1) * _VT
_NG = 8
_NW = 3


def _body(x_sr, table_r, w_r, b_ref, out_ref, emb_v, wbufs, gsems, wsems):
    j = pl.program_id(0)

    def _g_dma(i):
        return pltpu.make_async_copy(
            table_r.at[pl.ds(x_sr[i], 1), :],
            emb_v.at[pl.ds(i, 1), :],
            gsems.at[lax.rem(i, _NG)])

    def _w_dma(jj, width):
        return pltpu.make_async_copy(
            w_r.at[pl.ds(jj * _VT, width), :],
            wbufs.at[lax.rem(jj, _NW), pl.ds(0, width), :],
            wsems.at[lax.rem(jj, _NW)])

    @pl.when(j == 0)
    def _():
        for k in range(_NW - 1):
            _w_dma(k, _VT).start()
        lax.fori_loop(0, _TOKENS, lambda i, c: (_g_dma(i).start(), c)[1], 0,
                      unroll=8)
        lax.fori_loop(0, _TOKENS, lambda i, c: (_g_dma(i).wait(), c)[1], 0,
                      unroll=8)

    @pl.when(j + _NW - 1 < _NT - 1)
    def _():
        _w_dma(j + _NW - 1, _VT).start()

    @pl.when(j + _NW - 1 == _NT - 1)
    def _():
        _w_dma(_NT - 1, _TAIL).start()

    @pl.when(j < _NT - 1)
    def _():
        _w_dma(j, _VT).wait()

    @pl.when(j == _NT - 1)
    def _():
        _w_dma(_NT - 1, _TAIL).wait()

    emb = emb_v[...]
    acc = lax.dot_general(
        emb, wbufs[lax.rem(j, _NW)],
        dimension_numbers=(((1,), (1,)), ((), ())),
        preferred_element_type=jnp.float32,
    ) + b_ref[...]
    out_ref[...] = acc.reshape(_B, _L, _VT)


def kernel(x, embed_table, W, b):
    x_flat = x.reshape(-1).astype(jnp.int32)

    out = pl.pallas_call(
        _body,
        grid=(_NT,),
        in_specs=[
            pl.BlockSpec(memory_space=pltpu.SMEM),
            pl.BlockSpec(memory_space=pltpu.HBM),
            pl.BlockSpec(memory_space=pltpu.HBM),
            pl.BlockSpec((1, _VT), lambda j: (0, j)),
        ],
        out_specs=pl.BlockSpec((_B, _L, _VT), lambda j: (0, 0, j)),
        out_shape=jax.ShapeDtypeStruct((_B, _L, _VOCAB), jnp.float32),
        compiler_params=pltpu.CompilerParams(
            vmem_limit_bytes=100 * 1024 * 1024),
        scratch_shapes=[
            pltpu.VMEM((_TOKENS, _EMBED), jnp.float32),
            pltpu.VMEM((_NW, _VT, _EMBED), jnp.float32),
            pltpu.SemaphoreType.DMA((_NG,)),
            pltpu.SemaphoreType.DMA((_NW,)),
        ],
    )(x_flat, embed_table, W, b.reshape(1, _VOCAB))

    return out


# speedup vs baseline: 1.0129x; 1.0129x over previous
import jax
import jax.numpy as jnp
from jax import lax
from jax.experimental import pallas as pl
from jax.experimental.pallas import tpu as pltpu

_VOCAB = 100000
_EMBED = 64
_B = 16
_L = 16
_TOKENS = _B * _L
_VT = 8192
_NT = 13
_TAIL = _VOCAB - (_NT - 1) * _VT
_NG = 8
_NW = 3


def _body(x_sr, table_r, w_r, b_ref, out_ref, emb_v, wbufs, gsems, wsems):
    j = pl.program_id(0)

    def _g_dma(i):
        return pltpu.make_async_copy(
            table_r.at[pl.ds(x_sr[i], 1), :],
            emb_v.at[pl.ds(i, 1), :],
            gsems.at[lax.rem(i, _NG)])

    def _w_dma(jj, width):
        return pltpu.make_async_copy(
            w_r.at[pl.ds(jj * _VT, width), :],
            wbufs.at[lax.rem(jj, _NW), pl.ds(0, width), :],
            wsems.at[lax.rem(jj, _NW)])

    @pl.when(j == 0)
    def _():
        for k in range(_NW - 1):
            _w_dma(k, _VT).start()
        lax.fori_loop(0, _TOKENS, lambda i, c: (_g_dma(i).start(), c)[1], 0,
                      unroll=8)
        lax.fori_loop(0, _TOKENS, lambda i, c: (_g_dma(i).wait(), c)[1], 0,
                      unroll=8)

    @pl.when(j + _NW - 1 < _NT - 1)
    def _():
        _w_dma(j + _NW - 1, _VT).start()

    @pl.when(j + _NW - 1 == _NT - 1)
    def _():
        _w_dma(_NT - 1, _TAIL).start()

    @pl.when(j < _NT - 1)
    def _():
        _w_dma(j, _VT).wait()

    @pl.when(j == _NT - 1)
    def _():
        _w_dma(_NT - 1, _TAIL).wait()

    emb = emb_v[...]
    acc = lax.dot_general(
        emb, wbufs[lax.rem(j, _NW)],
        dimension_numbers=(((1,), (1,)), ((), ())),
        preferred_element_type=jnp.float32,
    ) + b_ref[...]
    out_ref[...] = acc.reshape(_B, _L, _VT)


def kernel(x, embed_table, W, b):
    x_flat = x.reshape(-1).astype(jnp.int32)

    out = pl.pallas_call(
        _body,
        grid=(_NT,),
        in_specs=[
            pl.BlockSpec(memory_space=pltpu.SMEM),
            pl.BlockSpec(memory_space=pltpu.HBM),
            pl.BlockSpec(memory_space=pltpu.HBM),
            pl.BlockSpec((1, _VT), lambda j: (0, j)),
        ],
        out_specs=pl.BlockSpec((_B, _L, _VT), lambda j: (0, 0, j)),
        out_shape=jax.ShapeDtypeStruct((_B, _L, _VOCAB), jnp.float32),
        compiler_params=pltpu.CompilerParams(
            vmem_limit_bytes=100 * 1024 * 1024),
        scratch_shapes=[
            pltpu.VMEM((_TOKENS, _EMBED), jnp.float32),
            pltpu.VMEM((_NW, _VT, _EMBED), jnp.float32),
            pltpu.SemaphoreType.DMA((_NG,)),
            pltpu.SemaphoreType.DMA((_NW,)),
        ],
    )(x_flat, embed_table, W, b.reshape(1, _VOCAB))

    return out


# speedup vs baseline: 1.0216x; 1.0086x over previous
import jax
import jax.numpy as jnp
from jax import lax
from jax.experimental import pallas as pl
from jax.experimental.pallas import tpu as pltpu

_VOCAB = 100000
_EMBED = 64
_B = 16
_L = 16
_TOKENS = _B * _L
_VT = 12800
_NT = 8
_TAIL = _VOCAB - (_NT - 1) * _VT
_NG = 8
_NW = 3


def _body(x_sr, table_r, w_r, b_ref, out_ref, emb_v, wbufs, gsems, wsems):
    j = pl.program_id(0)

    def _g_dma(i):
        return pltpu.make_async_copy(
            table_r.at[pl.ds(x_sr[i], 1), :],
            emb_v.at[pl.ds(i, 1), :],
            gsems.at[lax.rem(i, _NG)])

    def _w_dma(jj, width):
        return pltpu.make_async_copy(
            w_r.at[pl.ds(jj * _VT, width), :],
            wbufs.at[lax.rem(jj, _NW), pl.ds(0, width), :],
            wsems.at[lax.rem(jj, _NW)])

    @pl.when(j == 0)
    def _():
        for k in range(_NW - 1):
            _w_dma(k, _VT).start()
        lax.fori_loop(0, _TOKENS, lambda i, c: (_g_dma(i).start(), c)[1], 0,
                      unroll=8)
        lax.fori_loop(0, _TOKENS, lambda i, c: (_g_dma(i).wait(), c)[1], 0,
                      unroll=8)

    @pl.when(j + _NW - 1 < _NT - 1)
    def _():
        _w_dma(j + _NW - 1, _VT).start()

    @pl.when(j + _NW - 1 == _NT - 1)
    def _():
        _w_dma(_NT - 1, _TAIL).start()

    @pl.when(j < _NT - 1)
    def _():
        _w_dma(j, _VT).wait()

    @pl.when(j == _NT - 1)
    def _():
        _w_dma(_NT - 1, _TAIL).wait()

    emb = emb_v[...]
    acc = lax.dot_general(
        emb, wbufs[lax.rem(j, _NW)],
        dimension_numbers=(((1,), (1,)), ((), ())),
        preferred_element_type=jnp.float32,
    ) + b_ref[...]
    out_ref[...] = acc.reshape(_B, _L, _VT)


def kernel(x, embed_table, W, b):
    x_flat = x.reshape(-1).astype(jnp.int32)

    out = pl.pallas_call(
        _body,
        grid=(_NT,),
        in_specs=[
            pl.BlockSpec(memory_space=pltpu.SMEM),
            pl.BlockSpec(memory_space=pltpu.HBM),
            pl.BlockSpec(memory_space=pltpu.HBM),
            pl.BlockSpec((1, _VT), lambda j: (0, j)),
        ],
        out_specs=pl.BlockSpec((_B, _L, _VT), lambda j: (0, 0, j)),
        out_shape=jax.ShapeDtypeStruct((_B, _L, _VOCAB), jnp.float32),
        compiler_params=pltpu.CompilerParams(
            vmem_limit_bytes=100 * 1024 * 1024),
        scratch_shapes=[
            pltpu.VMEM((_TOKENS, _EMBED), jnp.float32),
            pltpu.VMEM((_NW, _VT, _EMBED), jnp.float32),
            pltpu.SemaphoreType.DMA((_NG,)),
            pltpu.SemaphoreType.DMA((_NW,)),
        ],
    )(x_flat, embed_table, W, b.reshape(1, _VOCAB))

    return out
